# baseline (speedup 1.0000x reference)
import functools

import jax
import jax.numpy as jnp
from jax import lax
from jax.experimental import pallas as pl
from jax.experimental.pallas import tpu as pltpu
from jax.experimental.pallas import tpu_sc as plsc

NUM_CORES = 2
NUM_SUBCORES = 16
NUM_WORKERS = NUM_CORES * NUM_SUBCORES
BATCH = 16384
DIM = 128
NJ2 = DIM // 32
BW = BATCH // NUM_WORKERS
CHUNK = 64
NCHUNK = BW // CHUNK

_ILV = plsc.PackFormat.INTERLEAVED


def _asbf16(x_i32):
    return plsc.bitcast(x_i32, jnp.bfloat16)


def _body(idx_hbm, e_hbm, rel_hbm, w_hbm, out_hbm,
          idxv, bufs0, bufs1, outb, sem0, sem1):
    wid = lax.axis_index("s") * NUM_CORES + lax.axis_index("c")

    pltpu.sync_copy(idx_hbm.at[wid], idxv)

    lanes = lax.iota(jnp.int32, 16)

    def issue(k, bufs, sem):
        pltpu.async_copy(e_hbm.at[idxv.at[0, k]], bufs[0], sem)
        pltpu.async_copy(e_hbm.at[idxv.at[1, k]], bufs[1], sem)
        pltpu.async_copy(rel_hbm.at[idxv.at[2, k]], bufs[2], sem)
        pltpu.async_copy(w_hbm.at[idxv.at[2, k]], bufs[3], sem)

    def drain(bufs, sem):
        for b in bufs:
            pltpu.make_async_copy(e_hbm.at[pl.ds(0, CHUNK)], b, sem).wait()

    def compute(k, bufs):
        hbuf, tbuf, rbuf, wbuf = bufs

        def group_body(g, carry2):
            outv = jnp.zeros((16,), jnp.float32)
            for rr in range(16):
                i = g * 16 + rr
                d = []
                w = []
                s1v = jnp.zeros((16,), jnp.float32)
                s2v = jnp.zeros((16,), jnp.float32)
                for j in range(NJ2):
                    sl = pl.ds(j * 16, 16)
                    h0, h1 = plsc.unpack(_asbf16(hbuf[i, sl]), format=_ILV)
                    t0, t1 = plsc.unpack(_asbf16(tbuf[i, sl]), format=_ILV)
                    w0, w1 = plsc.unpack(_asbf16(wbuf[i, sl]), format=_ILV)
                    d0 = h0 - t0
                    d1 = h1 - t1
                    d.append(d0)
                    d.append(d1)
                    w.append(w0)
                    w.append(w1)
                    s1v = s1v + d0 * w0 + d1 * w1
                    s2v = s2v + w0 * w0 + w1 * w1
                s1 = jnp.broadcast_to(jnp.sum(s1v), (16,))
                s2 = jnp.broadcast_to(jnp.sum(s2v), (16,))
                coeff = s1 / jnp.maximum(s2, 1e-24)
                accv = jnp.zeros((16,), jnp.float32)
                for j in range(NJ2):
                    r0, r1 = plsc.unpack(_asbf16(rbuf[i, pl.ds(j * 16, 16)]),
                                         format=_ILV)
                    accv = accv + jnp.abs(d[2 * j] + r0 - coeff * w[2 * j])
                    accv = accv + jnp.abs(d[2 * j + 1] + r1 - coeff * w[2 * j + 1])
                acc = jnp.broadcast_to(jnp.sum(accv), (16,))
                outv = jnp.where(lanes == rr, acc, outv)
            base = pl.multiple_of(k * CHUNK + g * 16, 16)
            outb[pl.ds(base, 16)] = outv
            return carry2

        lax.fori_loop(0, CHUNK // 16, group_body, 0)

    issue(0, bufs0, sem0)

    def pair_body(p, carry):
        k0 = 2 * p
        issue(k0 + 1, bufs1, sem1)
        drain(bufs0, sem0)
        compute(k0, bufs0)

        @pl.when(k0 + 2 < NCHUNK)
        def _():
            issue(k0 + 2, bufs0, sem0)

        drain(bufs1, sem1)
        compute(k0 + 1, bufs1)
        return carry

    lax.fori_loop(0, NCHUNK // 2, pair_body, 0)
    pltpu.sync_copy(outb, out_hbm.at[pl.ds(pl.multiple_of(wid * BW, 8), BW)])


@jax.jit
def kernel(h, r, t, E, R, W):
    mesh = plsc.VectorSubcoreMesh(core_axis_name="c", subcore_axis_name="s")
    buf = pltpu.VMEM((CHUNK, DIM // 2), jnp.int32)
    kfn = pl.kernel(
        _body,
        out_type=jax.ShapeDtypeStruct((BATCH,), jnp.float32),
        mesh=mesh,
        compiler_params=pltpu.CompilerParams(
            needs_layout_passes=False, use_tc_tiling_on_sc=False),
        scratch_types=[
            pltpu.VMEM((3, NCHUNK, CHUNK), jnp.int32),
            [buf, buf, buf, buf],
            [buf, buf, buf, buf],
            pltpu.VMEM((BW,), jnp.float32),
            pltpu.SemaphoreType.DMA,
            pltpu.SemaphoreType.DMA,
        ],
    )
    idx = jnp.stack([h, t, r]).reshape(3, NUM_WORKERS, NCHUNK, CHUNK)
    idx = idx.transpose(1, 0, 2, 3)

    def pack32(tbl):
        bf = tbl.astype(jnp.bfloat16)
        return lax.bitcast_convert_type(
            bf.reshape(tbl.shape[0], DIM // 2, 2), jnp.int32)

    return kfn(idx, pack32(E), pack32(R), pack32(W))


# speedup vs baseline: 7.0675x; 7.0675x over previous
import functools

import jax
import jax.numpy as jnp
from jax import lax
from jax.experimental import pallas as pl
from jax.experimental.pallas import tpu as pltpu
from jax.experimental.pallas import tpu_sc as plsc

NUM_CORES = 2
NUM_SUBCORES = 16
NUM_WORKERS = NUM_CORES * NUM_SUBCORES
BATCH = 16384
DIM = 128
NJ = DIM // 16
BW = BATCH // NUM_WORKERS
CHUNK = 64
NCHUNK = BW // CHUNK


def _body(idx_hbm, e_hbm, rel_hbm, w_hbm, out_hbm,
          idxv, bufs0, bufs1, outb, sem0, sem1):
    wid = lax.axis_index("s") * NUM_CORES + lax.axis_index("c")

    pltpu.sync_copy(idx_hbm.at[wid], idxv)

    lanes = lax.iota(jnp.int32, 16)

    def issue(k, bufs, sem):
        pltpu.async_copy(e_hbm.at[idxv.at[0, k]], bufs[0], sem)
        pltpu.async_copy(e_hbm.at[idxv.at[1, k]], bufs[1], sem)
        pltpu.async_copy(rel_hbm.at[idxv.at[2, k]], bufs[2], sem)
        pltpu.async_copy(w_hbm.at[idxv.at[2, k]], bufs[3], sem)

    def drain(bufs, sem):
        for b in bufs:
            pltpu.make_async_copy(e_hbm.at[pl.ds(0, CHUNK)], b, sem).wait()

    def compute(k, bufs):
        hbuf, tbuf, rbuf, wbuf = bufs

        def group_body(g, carry2):
            outvs = [jnp.zeros((16,), jnp.float32), jnp.zeros((16,), jnp.float32)]
            for rr in range(32):
                i = g * 32 + rr
                d = []
                w = []
                s1v = jnp.zeros((16,), jnp.float32)
                s2v = jnp.zeros((16,), jnp.float32)
                for j in range(NJ):
                    sl = pl.ds(j * 16, 16)
                    dv = hbuf[i, sl] - tbuf[i, sl]
                    wv = wbuf[i, sl]
                    d.append(dv)
                    w.append(wv)
                    s1v = s1v + dv * wv
                    s2v = s2v + wv * wv
                s1 = jnp.broadcast_to(jnp.sum(s1v), (16,))
                s2 = jnp.broadcast_to(jnp.sum(s2v), (16,))
                coeff = s1 / jnp.maximum(s2, 1e-24)
                accv = jnp.zeros((16,), jnp.float32)
                for j in range(NJ):
                    rv = rbuf[i, pl.ds(j * 16, 16)]
                    accv = accv + jnp.abs(d[j] + rv - coeff * w[j])
                acc = jnp.broadcast_to(jnp.sum(accv), (16,))
                outvs[rr // 16] = jnp.where(lanes == rr % 16, acc, outvs[rr // 16])
            base = pl.multiple_of(k * CHUNK + g * 32, 16)
            outb[pl.ds(base, 16)] = outvs[0]
            outb[pl.ds(base + 16, 16)] = outvs[1]
            return carry2

        lax.fori_loop(0, CHUNK // 32, group_body, 0)

    issue(0, bufs0, sem0)

    def pair_body(p, carry):
        k0 = 2 * p
        issue(k0 + 1, bufs1, sem1)
        drain(bufs0, sem0)
        compute(k0, bufs0)

        @pl.when(k0 + 2 < NCHUNK)
        def _():
            issue(k0 + 2, bufs0, sem0)

        drain(bufs1, sem1)
        compute(k0 + 1, bufs1)
        return carry

    lax.fori_loop(0, NCHUNK // 2, pair_body, 0)
    pltpu.sync_copy(outb, out_hbm.at[pl.ds(pl.multiple_of(wid * BW, 8), BW)])


@jax.jit
def kernel(h, r, t, E, R, W):
    mesh = plsc.VectorSubcoreMesh(core_axis_name="c", subcore_axis_name="s")
    buf = pltpu.VMEM((CHUNK, DIM), jnp.float32)
    kfn = pl.kernel(
        _body,
        out_type=jax.ShapeDtypeStruct((BATCH,), jnp.float32),
        mesh=mesh,
        compiler_params=pltpu.CompilerParams(needs_layout_passes=False),
        scratch_types=[
            pltpu.VMEM((3, NCHUNK, CHUNK), jnp.int32),
            [buf, buf, buf, buf],
            [buf, buf, buf, buf],
            pltpu.VMEM((BW,), jnp.float32),
            pltpu.SemaphoreType.DMA,
            pltpu.SemaphoreType.DMA,
        ],
    )
    idx = jnp.stack([h, t, r]).reshape(3, NUM_WORKERS, NCHUNK, CHUNK)
    idx = idx.transpose(1, 0, 2, 3)
    return kfn(idx, E, R, W)


# speedup vs baseline: 12.2040x; 1.7268x over previous
import functools

import jax
import jax.numpy as jnp
from jax import lax
from jax.experimental import pallas as pl
from jax.experimental.pallas import tpu as pltpu
from jax.experimental.pallas import tpu_sc as plsc

NUM_CORES = 2
NUM_SUBCORES = 16
NUM_WORKERS = NUM_CORES * NUM_SUBCORES
BATCH = 16384
DIM = 128
NJ = DIM // 16
BW = BATCH // NUM_WORKERS
CHUNK = 64
NCHUNK = BW // CHUNK


def _body(idx_hbm, e_hbm, rel_hbm, w_hbm, out_hbm,
          idxv, hb, tb, rb, wb, outb, sems):
    wid = lax.axis_index("s") * NUM_CORES + lax.axis_index("c")

    pltpu.sync_copy(idx_hbm.at[wid], idxv)

    lanes = lax.iota(jnp.int32, 16)

    def issue(k):
        par = k % 2
        sem = sems.at[par]
        pltpu.async_copy(e_hbm.at[idxv.at[0, k]], hb.at[par], sem)
        pltpu.async_copy(e_hbm.at[idxv.at[1, k]], tb.at[par], sem)
        pltpu.async_copy(rel_hbm.at[idxv.at[2, k]], rb.at[par], sem)
        pltpu.async_copy(w_hbm.at[idxv.at[2, k]], wb.at[par], sem)

    def drain(k):
        par = k % 2
        for b in (hb, tb, rb, wb):
            pltpu.make_async_copy(
                e_hbm.at[pl.ds(0, CHUNK)], b.at[par], sems.at[par]).wait()

    def compute(k):
        par = k % 2

        def group_body(g, carry2):
            outv = jnp.zeros((16,), jnp.float32)
            for rr in range(16):
                i = g * 16 + rr
                d = []
                w = []
                s1v = jnp.zeros((16,), jnp.float32)
                s2v = jnp.zeros((16,), jnp.float32)
                for j in range(NJ):
                    sl = pl.ds(j * 16, 16)
                    dv = hb[par, i, sl] - tb[par, i, sl]
                    wv = wb[par, i, sl]
                    d.append(dv)
                    w.append(wv)
                    s1v = s1v + dv * wv
                    s2v = s2v + wv * wv
                s1 = jnp.broadcast_to(jnp.sum(s1v), (16,))
                s2 = jnp.broadcast_to(jnp.sum(s2v), (16,))
                coeff = s1 / jnp.maximum(s2, 1e-24)
                accv = jnp.zeros((16,), jnp.float32)
                for j in range(NJ):
                    rv = rb[par, i, pl.ds(j * 16, 16)]
                    accv = accv + jnp.abs(d[j] + rv - coeff * w[j])
                acc = jnp.broadcast_to(jnp.sum(accv), (16,))
                outv = jnp.where(lanes == rr, acc, outv)
            base = pl.multiple_of(k * CHUNK + g * 16, 16)
            outb[pl.ds(base, 16)] = outv
            return carry2

        lax.fori_loop(0, CHUNK // 16, group_body, 0)

    issue(0)

    def chunk_body(k, carry):
        @pl.when(k + 1 < NCHUNK)
        def _():
            issue(k + 1)

        drain(k)
        compute(k)
        return carry

    lax.fori_loop(0, NCHUNK, chunk_body, 0)
    pltpu.sync_copy(outb, out_hbm.at[pl.ds(pl.multiple_of(wid * BW, 8), BW)])


@jax.jit
def kernel(h, r, t, E, R, W):
    mesh = plsc.VectorSubcoreMesh(core_axis_name="c", subcore_axis_name="s")
    buf = pltpu.VMEM((2, CHUNK, DIM), jnp.float32)
    kfn = pl.kernel(
        _body,
        out_type=jax.ShapeDtypeStruct((BATCH,), jnp.float32),
        mesh=mesh,
        compiler_params=pltpu.CompilerParams(needs_layout_passes=False),
        scratch_types=[
            pltpu.VMEM((3, NCHUNK, CHUNK), jnp.int32),
            buf, buf, buf, buf,
            pltpu.VMEM((BW,), jnp.float32),
            pltpu.SemaphoreType.DMA((2,)),
        ],
    )
    idx = jnp.stack([h, t, r]).reshape(3, NUM_WORKERS, NCHUNK, CHUNK)
    idx = idx.transpose(1, 0, 2, 3)
    return kfn(idx, E, R, W)
